# baseline (speedup 1.0000x reference)
import functools

import jax
import jax.numpy as jnp
from jax import lax
from jax.experimental import pallas as pl
from jax.experimental.pallas import tpu as pltpu
from jax.experimental.pallas import tpu_sc as plsc

N = 10000
D = 128
E = 320000
NC = 2
NS = 16
NW = NC * NS

CH = 128
NCHUNK = 80
EPW = NCHUNK * CH
EPW_DEG = E // NW
PAD_PW = EPW - EPW_DEG
ROWS_PT = 624
ROW_TAIL = N - NS * ROWS_PT
N_ACC = 10008
N_HIST = 10240

@functools.cache
def _sc_kernels():
    mesh = plsc.VectorSubcoreMesh(
        core_axis_name="c", subcore_axis_name="s", num_cores=NC, num_subcores=NS
    )
    sc_params = pltpu.CompilerParams(needs_layout_passes=False)
    deg_kernel = functools.partial(
        pl.kernel,
        out_type=jax.ShapeDtypeStruct((NW, N_HIST), jnp.float32),
        mesh=mesh,
        compiler_params=sc_params,
        scratch_types=[
            pltpu.VMEM((N_HIST,), jnp.float32),
            pltpu.VMEM((EPW_DEG,), jnp.int32),
        ],
    )(_deg_body)
    seg_kernel = functools.partial(
        pl.kernel,
        out_type=jax.ShapeDtypeStruct((NC, N, D), jnp.float32),
        mesh=mesh,
        compiler_params=sc_params,
        scratch_types=[
            pltpu.VMEM_SHARED((N_ACC, D), jnp.float32),
            pltpu.VMEM((NCHUNK, CH), jnp.int32),
            pltpu.VMEM((NCHUNK, CH), jnp.int32),
            pltpu.VMEM((CH, D), jnp.float32),
            pltpu.SemaphoreType.DMA,
        ],
    )(_seg_body)
    return deg_kernel, seg_kernel


def _deg_body(dst_hbm, out_hbm, hist, dstv):
    cid = lax.axis_index("c")
    sid = lax.axis_index("s")
    wid = sid * NC + cid

    zeros16 = jnp.zeros((16,), jnp.float32)

    def zbody(i, c):
        hist[pl.ds(i * 16, 16)] = zeros16
        return c

    lax.fori_loop(0, N_HIST // 16, zbody, 0)

    off = pl.multiple_of(wid * EPW_DEG, 8)
    pltpu.sync_copy(dst_hbm.at[pl.ds(off, EPW_DEG)], dstv)

    ones16 = jnp.ones((16,), jnp.float32)

    def body(i, c):
        idx = dstv[pl.ds(i * 16, 16)]
        plsc.addupdate_scatter(hist, [idx], ones16)
        return c

    lax.fori_loop(0, EPW_DEG // 16, body, 0)
    pltpu.sync_copy(hist, out_hbm.at[wid])


def _seg_body(xs_hbm, srcp_hbm, dstp_hbm, out_hbm, acc, srcv, dstv, rows, sem):
    cid = lax.axis_index("c")
    sid = lax.axis_index("s")
    wid = sid * NC + cid

    r0 = pl.multiple_of(sid * ROWS_PT, 8)
    pltpu.sync_copy(xs_hbm.at[pl.ds(r0, ROWS_PT)], acc.at[pl.ds(r0, ROWS_PT)])

    @pl.when(sid == 0)
    def _():
        t0 = NS * ROWS_PT
        pltpu.sync_copy(
            xs_hbm.at[pl.ds(t0, ROW_TAIL)], acc.at[pl.ds(t0, ROW_TAIL)]
        )

    pltpu.sync_copy(srcp_hbm.at[wid], srcv)
    pltpu.sync_copy(dstp_hbm.at[wid], dstv)
    plsc.subcore_barrier()

    def body(g, c):
        pltpu.async_copy(xs_hbm.at[srcv.at[g]], rows, sem).wait()
        pltpu.sync_copy(rows, acc.at[dstv.at[g]], add=True)
        return c

    lax.fori_loop(0, NCHUNK, body, 0)
    plsc.subcore_barrier()

    pltpu.sync_copy(
        acc.at[pl.ds(r0, ROWS_PT)], out_hbm.at[cid, pl.ds(r0, ROWS_PT)]
    )

    @pl.when(sid == 0)
    def _():
        t0 = NS * ROWS_PT
        pltpu.sync_copy(
            acc.at[pl.ds(t0, ROW_TAIL)], out_hbm.at[cid, pl.ds(t0, ROW_TAIL)]
        )


def _dinv_body(hists_ref, out_ref):
    deg = jnp.sum(hists_ref[...], axis=0) + 1.0
    out_ref[...] = lax.rsqrt(deg)


_dinv_call = pl.pallas_call(
    _dinv_body,
    out_shape=jax.ShapeDtypeStruct((N_HIST // 128, 128), jnp.float32),
)

BM = 2000


def _mm_body(x_ref, w_ref, dinv_ref, o_ref):
    xw = jnp.dot(x_ref[...], w_ref[...], preferred_element_type=jnp.float32)
    o_ref[...] = xw * dinv_ref[...]


_mm_call = pl.pallas_call(
    _mm_body,
    grid=(N // BM,),
    in_specs=[
        pl.BlockSpec((BM, D), lambda i: (i, 0)),
        pl.BlockSpec((D, D), lambda i: (0, 0)),
        pl.BlockSpec((BM, 1), lambda i: (i, 0)),
    ],
    out_specs=pl.BlockSpec((BM, D), lambda i: (i, 0)),
    out_shape=jax.ShapeDtypeStruct((N, D), jnp.float32),
)


def _post_body(p0_ref, p1_ref, xs_ref, dinv_ref, xin_ref, b_ref, g_ref, beta_ref, o_ref):
    agg = p0_ref[...] + p1_ref[...] - xs_ref[...]
    h = agg * dinv_ref[...] + b_ref[...]
    h = jnp.maximum(h, 0.0) + xin_ref[...]
    mu = jnp.mean(h, axis=-1, keepdims=True)
    d = h - mu
    var = jnp.mean(d * d, axis=-1, keepdims=True)
    o_ref[...] = d * lax.rsqrt(var + 1e-5) * g_ref[...] + beta_ref[...]


_post_call = pl.pallas_call(
    _post_body,
    grid=(N // BM,),
    in_specs=[
        pl.BlockSpec((BM, D), lambda i: (i, 0)),
        pl.BlockSpec((BM, D), lambda i: (i, 0)),
        pl.BlockSpec((BM, D), lambda i: (i, 0)),
        pl.BlockSpec((BM, 1), lambda i: (i, 0)),
        pl.BlockSpec((BM, D), lambda i: (i, 0)),
        pl.BlockSpec((1, D), lambda i: (0, 0)),
        pl.BlockSpec((1, D), lambda i: (0, 0)),
        pl.BlockSpec((1, D), lambda i: (0, 0)),
    ],
    out_specs=pl.BlockSpec((BM, D), lambda i: (i, 0)),
    out_shape=jax.ShapeDtypeStruct((N, D), jnp.float32),
)


@jax.jit
def _run(x, edge_index, Ws, bs, gammas, betas):
    src = edge_index[0]
    dst = edge_index[1]
    pad_src = jnp.zeros((NW, PAD_PW), jnp.int32)
    pad_dst = jnp.broadcast_to(
        N + (jnp.arange(PAD_PW, dtype=jnp.int32) % 8), (NW, PAD_PW)
    )
    srcp = jnp.concatenate([src.reshape(NW, EPW_DEG), pad_src], axis=1).reshape(
        NW, NCHUNK, CH
    )
    dstp = jnp.concatenate([dst.reshape(NW, EPW_DEG), pad_dst], axis=1).reshape(
        NW, NCHUNK, CH
    )

    deg_kernel, seg_kernel = _sc_kernels()
    hists = deg_kernel(dst)
    dinv2d = _dinv_call(hists.reshape(NW, N_HIST // 128, 128))
    dinv_col = dinv2d.reshape(-1)[:N][:, None]

    for i in range(3):
        xs = _mm_call(x, Ws[i], dinv_col)
        parts = seg_kernel(xs, srcp, dstp)
        x = _post_call(
            parts[0], parts[1], xs, dinv_col, x,
            bs[i][None, :], gammas[i][None, :], betas[i][None, :],
        )
    return x


def kernel(x, edge_index, Ws, bs, gammas, betas):
    return _run(x, edge_index, Ws, bs, gammas, betas)


# speedup vs baseline: 1.0002x; 1.0002x over previous
import functools

import jax
import jax.numpy as jnp
from jax import lax
from jax.experimental import pallas as pl
from jax.experimental.pallas import tpu as pltpu
from jax.experimental.pallas import tpu_sc as plsc

N = 10000
D = 128
E = 320000
NC = 2
NS = 16
NW = NC * NS

CH = 128
NCHUNK = 80
EPW = NCHUNK * CH
EPW_DEG = E // NW
PAD_PW = EPW - EPW_DEG
ROWS_PT = 624
ROW_TAIL = N - NS * ROWS_PT
N_ACC = 10248
N_HIST = 10240

@functools.cache
def _sc_kernels():
    mesh = plsc.VectorSubcoreMesh(
        core_axis_name="c", subcore_axis_name="s", num_cores=NC, num_subcores=NS
    )
    sc_params = pltpu.CompilerParams(needs_layout_passes=False)
    deg_kernel = functools.partial(
        pl.kernel,
        out_type=jax.ShapeDtypeStruct((NW, N_HIST), jnp.float32),
        mesh=mesh,
        compiler_params=sc_params,
        scratch_types=[
            pltpu.VMEM((N_HIST,), jnp.float32),
            pltpu.VMEM((EPW_DEG,), jnp.int32),
        ],
    )(_deg_body)
    seg_kernel = functools.partial(
        pl.kernel,
        out_type=jax.ShapeDtypeStruct((NC, N, D), jnp.float32),
        mesh=mesh,
        compiler_params=sc_params,
        scratch_types=[
            pltpu.VMEM_SHARED((N_ACC, D), jnp.float32),
            pltpu.VMEM((NCHUNK, CH), jnp.int32),
            pltpu.VMEM((NCHUNK, CH), jnp.int32),
            pltpu.VMEM((CH, D), jnp.float32),
            pltpu.SemaphoreType.DMA,
        ],
    )(_seg_body)
    return deg_kernel, seg_kernel


def _deg_body(dst_hbm, out_hbm, hist, dstv):
    cid = lax.axis_index("c")
    sid = lax.axis_index("s")
    wid = sid * NC + cid

    zeros16 = jnp.zeros((16,), jnp.float32)

    def zbody(i, c):
        hist[pl.ds(i * 16, 16)] = zeros16
        return c

    lax.fori_loop(0, N_HIST // 16, zbody, 0)

    off = pl.multiple_of(wid * EPW_DEG, 8)
    pltpu.sync_copy(dst_hbm.at[pl.ds(off, EPW_DEG)], dstv)

    ones16 = jnp.ones((16,), jnp.float32)

    def body(i, c):
        idx = dstv[pl.ds(i * 16, 16)]
        plsc.addupdate_scatter(hist, [idx], ones16)
        return c

    lax.fori_loop(0, EPW_DEG // 16, body, 0)
    pltpu.sync_copy(hist, out_hbm.at[wid])


def _seg_body(xs_hbm, srcp_hbm, dstp_hbm, out_hbm, acc, srcv, dstv, rows, sem):
    cid = lax.axis_index("c")
    sid = lax.axis_index("s")
    wid = sid * NC + cid

    r0 = pl.multiple_of(sid * ROWS_PT, 8)
    pltpu.sync_copy(xs_hbm.at[pl.ds(r0, ROWS_PT)], acc.at[pl.ds(r0, ROWS_PT)])

    @pl.when(sid == 0)
    def _():
        t0 = NS * ROWS_PT
        pltpu.sync_copy(
            xs_hbm.at[pl.ds(t0, ROW_TAIL)], acc.at[pl.ds(t0, ROW_TAIL)]
        )

    pltpu.sync_copy(srcp_hbm.at[wid], srcv)
    pltpu.sync_copy(dstp_hbm.at[wid], dstv)
    plsc.subcore_barrier()

    def body(g, c):
        pltpu.async_copy(xs_hbm.at[srcv.at[g]], rows, sem).wait()
        pltpu.sync_copy(rows, acc.at[dstv.at[g]], add=True)
        return c

    lax.fori_loop(0, NCHUNK, body, 0)
    plsc.subcore_barrier()

    pltpu.sync_copy(
        acc.at[pl.ds(r0, ROWS_PT)], out_hbm.at[cid, pl.ds(r0, ROWS_PT)]
    )

    @pl.when(sid == 0)
    def _():
        t0 = NS * ROWS_PT
        pltpu.sync_copy(
            acc.at[pl.ds(t0, ROW_TAIL)], out_hbm.at[cid, pl.ds(t0, ROW_TAIL)]
        )


def _dinv_body(hists_ref, out_ref):
    deg = jnp.sum(hists_ref[...], axis=0) + 1.0
    out_ref[...] = lax.rsqrt(deg)


_dinv_call = pl.pallas_call(
    _dinv_body,
    out_shape=jax.ShapeDtypeStruct((N_HIST // 128, 128), jnp.float32),
)

BM = 2000


def _mm_body(x_ref, w_ref, dinv_ref, o_ref):
    xw = jnp.dot(x_ref[...], w_ref[...], preferred_element_type=jnp.float32)
    o_ref[...] = xw * dinv_ref[...]


_mm_call = pl.pallas_call(
    _mm_body,
    grid=(N // BM,),
    in_specs=[
        pl.BlockSpec((BM, D), lambda i: (i, 0)),
        pl.BlockSpec((D, D), lambda i: (0, 0)),
        pl.BlockSpec((BM, 1), lambda i: (i, 0)),
    ],
    out_specs=pl.BlockSpec((BM, D), lambda i: (i, 0)),
    out_shape=jax.ShapeDtypeStruct((N, D), jnp.float32),
)


def _post_body(p0_ref, p1_ref, xs_ref, dinv_ref, xin_ref, b_ref, g_ref, beta_ref, o_ref):
    agg = p0_ref[...] + p1_ref[...] - xs_ref[...]
    h = agg * dinv_ref[...] + b_ref[...]
    h = jnp.maximum(h, 0.0) + xin_ref[...]
    mu = jnp.mean(h, axis=-1, keepdims=True)
    d = h - mu
    var = jnp.mean(d * d, axis=-1, keepdims=True)
    o_ref[...] = d * lax.rsqrt(var + 1e-5) * g_ref[...] + beta_ref[...]


_post_call = pl.pallas_call(
    _post_body,
    grid=(N // BM,),
    in_specs=[
        pl.BlockSpec((BM, D), lambda i: (i, 0)),
        pl.BlockSpec((BM, D), lambda i: (i, 0)),
        pl.BlockSpec((BM, D), lambda i: (i, 0)),
        pl.BlockSpec((BM, 1), lambda i: (i, 0)),
        pl.BlockSpec((BM, D), lambda i: (i, 0)),
        pl.BlockSpec((1, D), lambda i: (0, 0)),
        pl.BlockSpec((1, D), lambda i: (0, 0)),
        pl.BlockSpec((1, D), lambda i: (0, 0)),
    ],
    out_specs=pl.BlockSpec((BM, D), lambda i: (i, 0)),
    out_shape=jax.ShapeDtypeStruct((N, D), jnp.float32),
)


@jax.jit
def _run(x, edge_index, Ws, bs, gammas, betas):
    src = edge_index[0]
    dst = edge_index[1]
    pad_src = jnp.zeros((NW, PAD_PW), jnp.int32)
    pad_dst = jnp.broadcast_to(
        N + jnp.arange(PAD_PW, dtype=jnp.int32), (NW, PAD_PW)
    )
    srcp = jnp.concatenate([src.reshape(NW, EPW_DEG), pad_src], axis=1).reshape(
        NW, NCHUNK, CH
    )
    dstp = jnp.concatenate([dst.reshape(NW, EPW_DEG), pad_dst], axis=1).reshape(
        NW, NCHUNK, CH
    )

    deg_kernel, seg_kernel = _sc_kernels()
    hists = deg_kernel(dst)
    dinv2d = _dinv_call(hists.reshape(NW, N_HIST // 128, 128))
    dinv_col = dinv2d.reshape(-1)[:N][:, None]

    for i in range(3):
        xs = _mm_call(x, Ws[i], dinv_col)
        parts = seg_kernel(xs, srcp, dstp)
        x = _post_call(
            parts[0], parts[1], xs, dinv_col, x,
            bs[i][None, :], gammas[i][None, :], betas[i][None, :],
        )
    return x


def kernel(x, edge_index, Ws, bs, gammas, betas):
    return _run(x, edge_index, Ws, bs, gammas, betas)


# speedup vs baseline: 2.2759x; 2.2753x over previous
import functools

import jax
import jax.numpy as jnp
from jax import lax
from jax.experimental import pallas as pl
from jax.experimental.pallas import tpu as pltpu
from jax.experimental.pallas import tpu_sc as plsc

N = 10000
D = 128
E = 320000
NC = 2
NS = 16
NW = NC * NS

CH = 128
EPW_DEG = E // NW
NCHUNK = EPW_DEG // CH
CT = EPW_DEG - NCHUNK * CH
ROWS_PT = 624
ROW_TAIL = N - NS * ROWS_PT
N_ACC = N
N_HIST = 10240

@functools.cache
def _sc_kernels():
    mesh = plsc.VectorSubcoreMesh(
        core_axis_name="c", subcore_axis_name="s", num_cores=NC, num_subcores=NS
    )
    sc_params = pltpu.CompilerParams(needs_layout_passes=False)
    deg_kernel = functools.partial(
        pl.kernel,
        out_type=jax.ShapeDtypeStruct((NW, N_HIST), jnp.float32),
        mesh=mesh,
        compiler_params=sc_params,
        scratch_types=[
            pltpu.VMEM((N_HIST,), jnp.float32),
            pltpu.VMEM((EPW_DEG,), jnp.int32),
        ],
    )(_deg_body)
    seg_kernel = functools.partial(
        pl.kernel,
        out_type=jax.ShapeDtypeStruct((NC, N, D), jnp.float32),
        mesh=mesh,
        compiler_params=sc_params,
        scratch_types=[
            pltpu.VMEM_SHARED((N_ACC, D), jnp.float32),
            pltpu.VMEM((NCHUNK, CH), jnp.int32),
            pltpu.VMEM((NCHUNK, CH), jnp.int32),
            pltpu.VMEM((CT,), jnp.int32),
            pltpu.VMEM((CT,), jnp.int32),
            pltpu.VMEM((CH, D), jnp.float32),
            pltpu.SemaphoreType.DMA,
        ],
    )(_seg_body)
    return deg_kernel, seg_kernel


def _deg_body(dst_hbm, out_hbm, hist, dstv):
    cid = lax.axis_index("c")
    sid = lax.axis_index("s")
    wid = sid * NC + cid

    zeros16 = jnp.zeros((16,), jnp.float32)

    def zbody(i, c):
        hist[pl.ds(i * 16, 16)] = zeros16
        return c

    lax.fori_loop(0, N_HIST // 16, zbody, 0)

    off = pl.multiple_of(wid * EPW_DEG, 8)
    pltpu.sync_copy(dst_hbm.at[pl.ds(off, EPW_DEG)], dstv)

    ones16 = jnp.ones((16,), jnp.float32)

    def body(i, c):
        idx = dstv[pl.ds(i * 16, 16)]
        plsc.addupdate_scatter(hist, [idx], ones16)
        return c

    lax.fori_loop(0, EPW_DEG // 16, body, 0)
    pltpu.sync_copy(hist, out_hbm.at[wid])


def _seg_body(
    xs_hbm, srcp_hbm, dstp_hbm, srct_hbm, dstt_hbm, out_hbm,
    acc, srcv, dstv, srct, dstt, rows, sem,
):
    cid = lax.axis_index("c")
    sid = lax.axis_index("s")
    wid = sid * NC + cid

    r0 = pl.multiple_of(sid * ROWS_PT, 8)
    pltpu.sync_copy(xs_hbm.at[pl.ds(r0, ROWS_PT)], acc.at[pl.ds(r0, ROWS_PT)])

    @pl.when(sid == 0)
    def _():
        t0 = NS * ROWS_PT
        pltpu.sync_copy(
            xs_hbm.at[pl.ds(t0, ROW_TAIL)], acc.at[pl.ds(t0, ROW_TAIL)]
        )

    pltpu.sync_copy(srcp_hbm.at[wid], srcv)
    pltpu.sync_copy(dstp_hbm.at[wid], dstv)
    pltpu.sync_copy(srct_hbm.at[wid], srct)
    pltpu.sync_copy(dstt_hbm.at[wid], dstt)
    plsc.subcore_barrier()

    def body(g, c):
        pltpu.async_copy(xs_hbm.at[srcv.at[g]], rows, sem).wait()
        pltpu.sync_copy(rows, acc.at[dstv.at[g]], add=True)
        return c

    lax.fori_loop(0, NCHUNK, body, 0)
    pltpu.async_copy(xs_hbm.at[srct], rows.at[pl.ds(0, CT)], sem).wait()
    pltpu.sync_copy(rows.at[pl.ds(0, CT)], acc.at[dstt], add=True)
    plsc.subcore_barrier()

    pltpu.sync_copy(
        acc.at[pl.ds(r0, ROWS_PT)], out_hbm.at[cid, pl.ds(r0, ROWS_PT)]
    )

    @pl.when(sid == 0)
    def _():
        t0 = NS * ROWS_PT
        pltpu.sync_copy(
            acc.at[pl.ds(t0, ROW_TAIL)], out_hbm.at[cid, pl.ds(t0, ROW_TAIL)]
        )


def _dinv_body(hists_ref, out_ref):
    deg = jnp.sum(hists_ref[...], axis=0) + 1.0
    out_ref[...] = lax.rsqrt(deg)


_dinv_call = pl.pallas_call(
    _dinv_body,
    out_shape=jax.ShapeDtypeStruct((N_HIST // 128, 128), jnp.float32),
)

BM = 2000


def _mm_body(x_ref, w_ref, dinv_ref, o_ref):
    xw = jnp.dot(x_ref[...], w_ref[...], preferred_element_type=jnp.float32)
    o_ref[...] = xw * dinv_ref[...]


_mm_call = pl.pallas_call(
    _mm_body,
    grid=(N // BM,),
    in_specs=[
        pl.BlockSpec((BM, D), lambda i: (i, 0)),
        pl.BlockSpec((D, D), lambda i: (0, 0)),
        pl.BlockSpec((BM, 1), lambda i: (i, 0)),
    ],
    out_specs=pl.BlockSpec((BM, D), lambda i: (i, 0)),
    out_shape=jax.ShapeDtypeStruct((N, D), jnp.float32),
)


def _post_body(p0_ref, p1_ref, xs_ref, dinv_ref, xin_ref, b_ref, g_ref, beta_ref, o_ref):
    agg = p0_ref[...] + p1_ref[...] - xs_ref[...]
    h = agg * dinv_ref[...] + b_ref[...]
    h = jnp.maximum(h, 0.0) + xin_ref[...]
    mu = jnp.mean(h, axis=-1, keepdims=True)
    d = h - mu
    var = jnp.mean(d * d, axis=-1, keepdims=True)
    o_ref[...] = d * lax.rsqrt(var + 1e-5) * g_ref[...] + beta_ref[...]


_post_call = pl.pallas_call(
    _post_body,
    grid=(N // BM,),
    in_specs=[
        pl.BlockSpec((BM, D), lambda i: (i, 0)),
        pl.BlockSpec((BM, D), lambda i: (i, 0)),
        pl.BlockSpec((BM, D), lambda i: (i, 0)),
        pl.BlockSpec((BM, 1), lambda i: (i, 0)),
        pl.BlockSpec((BM, D), lambda i: (i, 0)),
        pl.BlockSpec((1, D), lambda i: (0, 0)),
        pl.BlockSpec((1, D), lambda i: (0, 0)),
        pl.BlockSpec((1, D), lambda i: (0, 0)),
    ],
    out_specs=pl.BlockSpec((BM, D), lambda i: (i, 0)),
    out_shape=jax.ShapeDtypeStruct((N, D), jnp.float32),
)


@jax.jit
def _run(x, edge_index, Ws, bs, gammas, betas):
    src = edge_index[0]
    dst = edge_index[1]
    srcw = src.reshape(NW, EPW_DEG)
    dstw = dst.reshape(NW, EPW_DEG)
    srcp = srcw[:, : NCHUNK * CH].reshape(NW, NCHUNK, CH)
    dstp = dstw[:, : NCHUNK * CH].reshape(NW, NCHUNK, CH)
    srct = srcw[:, NCHUNK * CH :]
    dstt = dstw[:, NCHUNK * CH :]

    deg_kernel, seg_kernel = _sc_kernels()
    hists = deg_kernel(dst)
    dinv2d = _dinv_call(hists.reshape(NW, N_HIST // 128, 128))
    dinv_col = dinv2d.reshape(-1)[:N][:, None]

    for i in range(3):
        xs = _mm_call(x, Ws[i], dinv_col)
        parts = seg_kernel(xs, srcp, dstp, srct, dstt)
        x = _post_call(
            parts[0], parts[1], xs, dinv_col, x,
            bs[i][None, :], gammas[i][None, :], betas[i][None, :],
        )
    return x


def kernel(x, edge_index, Ws, bs, gammas, betas):
    return _run(x, edge_index, Ws, bs, gammas, betas)


# speedup vs baseline: 3.4013x; 1.4945x over previous
import functools

import jax
import jax.numpy as jnp
from jax import lax
from jax.experimental import pallas as pl
from jax.experimental.pallas import tpu as pltpu
from jax.experimental.pallas import tpu_sc as plsc

N = 10000
D = 128
E = 320000
NC = 2
NS = 16
NW = NC * NS

CH = 64
EPW_DEG = E // NW
NCHUNK = EPW_DEG // CH
CT = EPW_DEG - NCHUNK * CH
NSLOT = 5
ROWS_PT = 624
ROW_TAIL = N - NS * ROWS_PT
N_ACC = N
N_HIST = 10240

@functools.cache
def _sc_kernels():
    mesh = plsc.VectorSubcoreMesh(
        core_axis_name="c", subcore_axis_name="s", num_cores=NC, num_subcores=NS
    )
    sc_params = pltpu.CompilerParams(needs_layout_passes=False)
    deg_kernel = functools.partial(
        pl.kernel,
        out_type=jax.ShapeDtypeStruct((NW, N_HIST), jnp.float32),
        mesh=mesh,
        compiler_params=sc_params,
        scratch_types=[
            pltpu.VMEM((N_HIST,), jnp.float32),
            pltpu.VMEM((EPW_DEG,), jnp.int32),
        ],
    )(_deg_body)
    seg_kernel = functools.partial(
        pl.kernel,
        out_type=jax.ShapeDtypeStruct((NC, N, D), jnp.float32),
        mesh=mesh,
        compiler_params=sc_params,
        scratch_types=[
            pltpu.VMEM_SHARED((N_ACC, D), jnp.float32),
            pltpu.VMEM((CT,), jnp.int32),
            pltpu.VMEM((CT,), jnp.int32),
        ]
        + [pltpu.VMEM((CH,), jnp.int32) for _ in range(2 * NSLOT)]
        + [pltpu.VMEM((CH, D), jnp.float32) for _ in range(NSLOT)]
        + [pltpu.SemaphoreType.DMA for _ in range(3 * NSLOT)],
    )(_seg_body)
    return deg_kernel, seg_kernel


def _deg_body(dst_hbm, out_hbm, hist, dstv):
    cid = lax.axis_index("c")
    sid = lax.axis_index("s")
    wid = sid * NC + cid

    zeros16 = jnp.zeros((16,), jnp.float32)

    def zbody(i, c):
        hist[pl.ds(i * 16, 16)] = zeros16
        return c

    lax.fori_loop(0, N_HIST // 16, zbody, 0)

    off = pl.multiple_of(wid * EPW_DEG, 8)
    pltpu.sync_copy(dst_hbm.at[pl.ds(off, EPW_DEG)], dstv)

    ones16 = jnp.ones((16,), jnp.float32)

    def body(i, c):
        idx = dstv[pl.ds(i * 16, 16)]
        plsc.addupdate_scatter(hist, [idx], ones16)
        return c

    lax.fori_loop(0, EPW_DEG // 16, body, 0)
    pltpu.sync_copy(hist, out_hbm.at[wid])


def _seg_body(xs_hbm, srcp_hbm, dstp_hbm, srct_hbm, dstt_hbm, out_hbm,
              acc, srct, dstt, *rest):
    sidx = rest[:NSLOT]
    didx = rest[NSLOT : 2 * NSLOT]
    rows = rest[2 * NSLOT : 3 * NSLOT]
    isem = rest[3 * NSLOT : 4 * NSLOT]
    gsem = rest[4 * NSLOT : 5 * NSLOT]
    ssem = rest[5 * NSLOT : 6 * NSLOT]
    cid = lax.axis_index("c")
    sid = lax.axis_index("s")
    wid = sid * NC + cid

    r0 = pl.multiple_of(sid * ROWS_PT, 8)
    pltpu.sync_copy(xs_hbm.at[pl.ds(r0, ROWS_PT)], acc.at[pl.ds(r0, ROWS_PT)])

    @pl.when(sid == 0)
    def _():
        t0 = NS * ROWS_PT
        pltpu.sync_copy(
            xs_hbm.at[pl.ds(t0, ROW_TAIL)], acc.at[pl.ds(t0, ROW_TAIL)]
        )

    pltpu.sync_copy(srct_hbm.at[wid], srct)
    pltpu.sync_copy(dstt_hbm.at[wid], dstt)
    plsc.subcore_barrier()

    def fire_idx(g, b):
        pltpu.async_copy(srcp_hbm.at[wid, g], sidx[b], isem[b])
        pltpu.async_copy(dstp_hbm.at[wid, g], didx[b], isem[b])

    def wait_idx(g, b):
        pltpu.make_async_copy(srcp_hbm.at[wid, g], sidx[b], isem[b]).wait()
        pltpu.make_async_copy(dstp_hbm.at[wid, g], didx[b], isem[b]).wait()

    def fire_gather(b):
        pltpu.async_copy(xs_hbm.at[sidx[b]], rows[b], gsem[b])

    def wait_gather(b):
        pltpu.make_async_copy(xs_hbm.at[sidx[b]], rows[b], gsem[b]).wait()

    def fire_scatter(b):
        pltpu.async_copy(rows[b], acc.at[didx[b]], ssem[b], add=True)

    def wait_scatter(b):
        pltpu.make_async_copy(rows[b], acc.at[didx[b]], ssem[b]).wait()

    def step(g, pos):
        b_a = pos % NSLOT
        if pos >= NSLOT:
            wait_scatter(b_a)
        fire_idx(g, b_a)
        if pos >= 1:
            b_b = (pos - 1) % NSLOT
            wait_idx(g - 1, b_b)
            fire_gather(b_b)
        if pos >= 3:
            b_c = (pos - 3) % NSLOT
            wait_gather(b_c)
            fire_scatter(b_c)

    for p in range(NSLOT):
        step(p, p)

    KS = (NCHUNK - NSLOT) // NSLOT

    def blk(j, c):
        g0 = NSLOT + NSLOT * j
        for b in range(NSLOT):
            step(g0 + b, NSLOT + b)
        return c

    lax.fori_loop(0, KS, blk, 0)

    for g in range(NSLOT + NSLOT * KS, NCHUNK):
        step(g, NSLOT + g % NSLOT)
    for g in range(NCHUNK, NCHUNK + 3):
        b_b = (g - 1) % NSLOT
        if g - 1 < NCHUNK:
            wait_idx(g - 1, b_b)
            fire_gather(b_b)
        b_c = (g - 3) % NSLOT
        if g - 3 < NCHUNK:
            wait_gather(b_c)
            fire_scatter(b_c)
    for c in range(NCHUNK - NSLOT, NCHUNK):
        wait_scatter(c % NSLOT)
    pltpu.async_copy(xs_hbm.at[srct], rows[0].at[pl.ds(0, CT)], gsem[0]).wait()
    pltpu.sync_copy(rows[0].at[pl.ds(0, CT)], acc.at[dstt], add=True)
    plsc.subcore_barrier()

    pltpu.sync_copy(
        acc.at[pl.ds(r0, ROWS_PT)], out_hbm.at[cid, pl.ds(r0, ROWS_PT)]
    )

    @pl.when(sid == 0)
    def _():
        t0 = NS * ROWS_PT
        pltpu.sync_copy(
            acc.at[pl.ds(t0, ROW_TAIL)], out_hbm.at[cid, pl.ds(t0, ROW_TAIL)]
        )


def _dinv_body(hists_ref, out_ref):
    deg = jnp.sum(hists_ref[...], axis=0) + 1.0
    out_ref[...] = lax.rsqrt(deg)


_dinv_call = pl.pallas_call(
    _dinv_body,
    out_shape=jax.ShapeDtypeStruct((N_HIST // 128, 128), jnp.float32),
)

BM = 2000


def _mm_body(x_ref, w_ref, dinv_ref, o_ref):
    xw = jnp.dot(x_ref[...], w_ref[...], preferred_element_type=jnp.float32)
    o_ref[...] = xw * dinv_ref[...]


_mm_call = pl.pallas_call(
    _mm_body,
    grid=(N // BM,),
    in_specs=[
        pl.BlockSpec((BM, D), lambda i: (i, 0)),
        pl.BlockSpec((D, D), lambda i: (0, 0)),
        pl.BlockSpec((BM, 1), lambda i: (i, 0)),
    ],
    out_specs=pl.BlockSpec((BM, D), lambda i: (i, 0)),
    out_shape=jax.ShapeDtypeStruct((N, D), jnp.float32),
)


def _post_body(p0_ref, p1_ref, xs_ref, dinv_ref, xin_ref, b_ref, g_ref, beta_ref, o_ref):
    agg = p0_ref[...] + p1_ref[...] - xs_ref[...]
    h = agg * dinv_ref[...] + b_ref[...]
    h = jnp.maximum(h, 0.0) + xin_ref[...]
    mu = jnp.mean(h, axis=-1, keepdims=True)
    d = h - mu
    var = jnp.mean(d * d, axis=-1, keepdims=True)
    o_ref[...] = d * lax.rsqrt(var + 1e-5) * g_ref[...] + beta_ref[...]


_post_call = pl.pallas_call(
    _post_body,
    grid=(N // BM,),
    in_specs=[
        pl.BlockSpec((BM, D), lambda i: (i, 0)),
        pl.BlockSpec((BM, D), lambda i: (i, 0)),
        pl.BlockSpec((BM, D), lambda i: (i, 0)),
        pl.BlockSpec((BM, 1), lambda i: (i, 0)),
        pl.BlockSpec((BM, D), lambda i: (i, 0)),
        pl.BlockSpec((1, D), lambda i: (0, 0)),
        pl.BlockSpec((1, D), lambda i: (0, 0)),
        pl.BlockSpec((1, D), lambda i: (0, 0)),
    ],
    out_specs=pl.BlockSpec((BM, D), lambda i: (i, 0)),
    out_shape=jax.ShapeDtypeStruct((N, D), jnp.float32),
)


@jax.jit
def _run(x, edge_index, Ws, bs, gammas, betas):
    src = edge_index[0]
    dst = edge_index[1]
    srcw = src.reshape(NW, EPW_DEG)
    dstw = dst.reshape(NW, EPW_DEG)
    srcp = srcw[:, : NCHUNK * CH].reshape(NW, NCHUNK, CH)
    dstp = dstw[:, : NCHUNK * CH].reshape(NW, NCHUNK, CH)
    srct = srcw[:, NCHUNK * CH :]
    dstt = dstw[:, NCHUNK * CH :]

    deg_kernel, seg_kernel = _sc_kernels()
    hists = deg_kernel(dst)
    dinv2d = _dinv_call(hists.reshape(NW, N_HIST // 128, 128))
    dinv_col = dinv2d.reshape(-1)[:N][:, None]

    for i in range(3):
        xs = _mm_call(x, Ws[i], dinv_col)
        parts = seg_kernel(xs, srcp, dstp, srct, dstt)
        x = _post_call(
            parts[0], parts[1], xs, dinv_col, x,
            bs[i][None, :], gammas[i][None, :], betas[i][None, :],
        )
    return x


def kernel(x, edge_index, Ws, bs, gammas, betas):
    return _run(x, edge_index, Ws, bs, gammas, betas)


# speedup vs baseline: 3.4125x; 1.0033x over previous
import functools

import jax
import jax.numpy as jnp
from jax import lax
from jax.experimental import pallas as pl
from jax.experimental.pallas import tpu as pltpu
from jax.experimental.pallas import tpu_sc as plsc

N = 10000
D = 128
E = 320000
NC = 2
NS = 16
NW = NC * NS

CH = 64
EPW_DEG = E // NW
NCHUNK = EPW_DEG // CH
CT = EPW_DEG - NCHUNK * CH
NSLOT = 6
LAG = 3
ROWS_PT = 624
ROW_TAIL = N - NS * ROWS_PT
N_ACC = N
N_HIST = 10240

@functools.cache
def _sc_kernels():
    mesh = plsc.VectorSubcoreMesh(
        core_axis_name="c", subcore_axis_name="s", num_cores=NC, num_subcores=NS
    )
    sc_params = pltpu.CompilerParams(needs_layout_passes=False)
    deg_kernel = functools.partial(
        pl.kernel,
        out_type=jax.ShapeDtypeStruct((NW, N_HIST), jnp.float32),
        mesh=mesh,
        compiler_params=sc_params,
        scratch_types=[
            pltpu.VMEM((N_HIST,), jnp.float32),
            pltpu.VMEM((EPW_DEG,), jnp.int32),
        ],
    )(_deg_body)
    seg_kernel = functools.partial(
        pl.kernel,
        out_type=jax.ShapeDtypeStruct((NC, N, D), jnp.float32),
        mesh=mesh,
        compiler_params=sc_params,
        scratch_types=[
            pltpu.VMEM_SHARED((N_ACC, D), jnp.float32),
            pltpu.VMEM((CT,), jnp.int32),
            pltpu.VMEM((CT,), jnp.int32),
        ]
        + [pltpu.VMEM((CH,), jnp.int32) for _ in range(2 * NSLOT)]
        + [pltpu.VMEM((CH, D), jnp.float32) for _ in range(NSLOT)]
        + [pltpu.SemaphoreType.DMA for _ in range(3 * NSLOT)],
    )(_seg_body)
    return deg_kernel, seg_kernel


def _deg_body(dst_hbm, out_hbm, hist, dstv):
    cid = lax.axis_index("c")
    sid = lax.axis_index("s")
    wid = sid * NC + cid

    zeros16 = jnp.zeros((16,), jnp.float32)

    def zbody(i, c):
        hist[pl.ds(i * 16, 16)] = zeros16
        return c

    lax.fori_loop(0, N_HIST // 16, zbody, 0)

    off = pl.multiple_of(wid * EPW_DEG, 8)
    pltpu.sync_copy(dst_hbm.at[pl.ds(off, EPW_DEG)], dstv)

    ones16 = jnp.ones((16,), jnp.float32)

    def body(i, c):
        idx = dstv[pl.ds(i * 16, 16)]
        plsc.addupdate_scatter(hist, [idx], ones16)
        return c

    lax.fori_loop(0, EPW_DEG // 16, body, 0)
    pltpu.sync_copy(hist, out_hbm.at[wid])


def _seg_body(xs_hbm, srcp_hbm, dstp_hbm, srct_hbm, dstt_hbm, out_hbm,
              acc, srct, dstt, *rest):
    sidx = rest[:NSLOT]
    didx = rest[NSLOT : 2 * NSLOT]
    rows = rest[2 * NSLOT : 3 * NSLOT]
    isem = rest[3 * NSLOT : 4 * NSLOT]
    gsem = rest[4 * NSLOT : 5 * NSLOT]
    ssem = rest[5 * NSLOT : 6 * NSLOT]
    cid = lax.axis_index("c")
    sid = lax.axis_index("s")
    wid = sid * NC + cid

    r0 = pl.multiple_of(sid * ROWS_PT, 8)
    pltpu.sync_copy(xs_hbm.at[pl.ds(r0, ROWS_PT)], acc.at[pl.ds(r0, ROWS_PT)])

    @pl.when(sid == 0)
    def _():
        t0 = NS * ROWS_PT
        pltpu.sync_copy(
            xs_hbm.at[pl.ds(t0, ROW_TAIL)], acc.at[pl.ds(t0, ROW_TAIL)]
        )

    pltpu.sync_copy(srct_hbm.at[wid], srct)
    pltpu.sync_copy(dstt_hbm.at[wid], dstt)
    plsc.subcore_barrier()

    def fire_idx(g, b):
        pltpu.async_copy(srcp_hbm.at[wid, g], sidx[b], isem[b])
        pltpu.async_copy(dstp_hbm.at[wid, g], didx[b], isem[b])

    def wait_idx(g, b):
        pltpu.make_async_copy(srcp_hbm.at[wid, g], sidx[b], isem[b]).wait()
        pltpu.make_async_copy(dstp_hbm.at[wid, g], didx[b], isem[b]).wait()

    def fire_gather(b):
        pltpu.async_copy(xs_hbm.at[sidx[b]], rows[b], gsem[b])

    def wait_gather(b):
        pltpu.make_async_copy(xs_hbm.at[sidx[b]], rows[b], gsem[b]).wait()

    def fire_scatter(b):
        pltpu.async_copy(rows[b], acc.at[didx[b]], ssem[b], add=True)

    def wait_scatter(b):
        pltpu.make_async_copy(rows[b], acc.at[didx[b]], ssem[b]).wait()

    def step(g, pos):
        b_a = pos % NSLOT
        if pos >= NSLOT:
            wait_scatter(b_a)
        fire_idx(g, b_a)
        if pos >= 1:
            b_b = (pos - 1) % NSLOT
            wait_idx(g - 1, b_b)
            fire_gather(b_b)
        if pos >= LAG:
            b_c = (pos - LAG) % NSLOT
            wait_gather(b_c)
            fire_scatter(b_c)

    for p in range(NSLOT):
        step(p, p)

    KS = (NCHUNK - NSLOT) // NSLOT

    def blk(j, c):
        g0 = NSLOT + NSLOT * j
        for b in range(NSLOT):
            step(g0 + b, NSLOT + b)
        return c

    lax.fori_loop(0, KS, blk, 0)

    for g in range(NSLOT + NSLOT * KS, NCHUNK):
        step(g, NSLOT + g % NSLOT)
    for g in range(NCHUNK, NCHUNK + LAG):
        b_b = (g - 1) % NSLOT
        if g - 1 < NCHUNK:
            wait_idx(g - 1, b_b)
            fire_gather(b_b)
        b_c = (g - LAG) % NSLOT
        if g - LAG < NCHUNK:
            wait_gather(b_c)
            fire_scatter(b_c)
    for c in range(NCHUNK - NSLOT, NCHUNK):
        wait_scatter(c % NSLOT)
    pltpu.async_copy(xs_hbm.at[srct], rows[0].at[pl.ds(0, CT)], gsem[0]).wait()
    pltpu.sync_copy(rows[0].at[pl.ds(0, CT)], acc.at[dstt], add=True)
    plsc.subcore_barrier()

    pltpu.sync_copy(
        acc.at[pl.ds(r0, ROWS_PT)], out_hbm.at[cid, pl.ds(r0, ROWS_PT)]
    )

    @pl.when(sid == 0)
    def _():
        t0 = NS * ROWS_PT
        pltpu.sync_copy(
            acc.at[pl.ds(t0, ROW_TAIL)], out_hbm.at[cid, pl.ds(t0, ROW_TAIL)]
        )


def _dinv_body(hists_ref, out_ref):
    deg = jnp.sum(hists_ref[...], axis=0) + 1.0
    out_ref[...] = lax.rsqrt(deg)


_dinv_call = pl.pallas_call(
    _dinv_body,
    out_shape=jax.ShapeDtypeStruct((N_HIST // 128, 128), jnp.float32),
)

BM = 2000


def _mm_body(x_ref, w_ref, dinv_ref, o_ref):
    xw = jnp.dot(x_ref[...], w_ref[...], preferred_element_type=jnp.float32)
    o_ref[...] = xw * dinv_ref[...]


_mm_call = pl.pallas_call(
    _mm_body,
    grid=(N // BM,),
    in_specs=[
        pl.BlockSpec((BM, D), lambda i: (i, 0)),
        pl.BlockSpec((D, D), lambda i: (0, 0)),
        pl.BlockSpec((BM, 1), lambda i: (i, 0)),
    ],
    out_specs=pl.BlockSpec((BM, D), lambda i: (i, 0)),
    out_shape=jax.ShapeDtypeStruct((N, D), jnp.float32),
)


def _post_body(p0_ref, p1_ref, xs_ref, dinv_ref, xin_ref, b_ref, g_ref, beta_ref, o_ref):
    agg = p0_ref[...] + p1_ref[...] - xs_ref[...]
    h = agg * dinv_ref[...] + b_ref[...]
    h = jnp.maximum(h, 0.0) + xin_ref[...]
    mu = jnp.mean(h, axis=-1, keepdims=True)
    d = h - mu
    var = jnp.mean(d * d, axis=-1, keepdims=True)
    o_ref[...] = d * lax.rsqrt(var + 1e-5) * g_ref[...] + beta_ref[...]


_post_call = pl.pallas_call(
    _post_body,
    grid=(N // BM,),
    in_specs=[
        pl.BlockSpec((BM, D), lambda i: (i, 0)),
        pl.BlockSpec((BM, D), lambda i: (i, 0)),
        pl.BlockSpec((BM, D), lambda i: (i, 0)),
        pl.BlockSpec((BM, 1), lambda i: (i, 0)),
        pl.BlockSpec((BM, D), lambda i: (i, 0)),
        pl.BlockSpec((1, D), lambda i: (0, 0)),
        pl.BlockSpec((1, D), lambda i: (0, 0)),
        pl.BlockSpec((1, D), lambda i: (0, 0)),
    ],
    out_specs=pl.BlockSpec((BM, D), lambda i: (i, 0)),
    out_shape=jax.ShapeDtypeStruct((N, D), jnp.float32),
)


@jax.jit
def _run(x, edge_index, Ws, bs, gammas, betas):
    src = edge_index[0]
    dst = edge_index[1]
    srcw = src.reshape(NW, EPW_DEG)
    dstw = dst.reshape(NW, EPW_DEG)
    srcp = srcw[:, : NCHUNK * CH].reshape(NW, NCHUNK, CH)
    dstp = dstw[:, : NCHUNK * CH].reshape(NW, NCHUNK, CH)
    srct = srcw[:, NCHUNK * CH :]
    dstt = dstw[:, NCHUNK * CH :]

    deg_kernel, seg_kernel = _sc_kernels()
    hists = deg_kernel(dst)
    dinv2d = _dinv_call(hists.reshape(NW, N_HIST // 128, 128))
    dinv_col = dinv2d.reshape(-1)[:N][:, None]

    for i in range(3):
        xs = _mm_call(x, Ws[i], dinv_col)
        parts = seg_kernel(xs, srcp, dstp, srct, dstt)
        x = _post_call(
            parts[0], parts[1], xs, dinv_col, x,
            bs[i][None, :], gammas[i][None, :], betas[i][None, :],
        )
    return x


def kernel(x, edge_index, Ws, bs, gammas, betas):
    return _run(x, edge_index, Ws, bs, gammas, betas)


# speedup vs baseline: 3.4819x; 1.0203x over previous
import functools

import jax
import jax.numpy as jnp
from jax import lax
from jax.experimental import pallas as pl
from jax.experimental.pallas import tpu as pltpu
from jax.experimental.pallas import tpu_sc as plsc

N = 10000
D = 128
E = 320000
NC = 2
NS = 16
NW = NC * NS

CH = 64
EPW_DEG = E // NW
NCHUNK = EPW_DEG // CH
CT = EPW_DEG - NCHUNK * CH
NSLOT = 6
LAG = 4
ROWS_PT = 624
ROW_TAIL = N - NS * ROWS_PT
N_ACC = N
N_HIST = 10240

@functools.cache
def _sc_kernels():
    mesh = plsc.VectorSubcoreMesh(
        core_axis_name="c", subcore_axis_name="s", num_cores=NC, num_subcores=NS
    )
    sc_params = pltpu.CompilerParams(needs_layout_passes=False)
    deg_kernel = functools.partial(
        pl.kernel,
        out_type=jax.ShapeDtypeStruct((NW, N_HIST), jnp.float32),
        mesh=mesh,
        compiler_params=sc_params,
        scratch_types=[
            pltpu.VMEM((N_HIST,), jnp.float32),
            pltpu.VMEM((EPW_DEG,), jnp.int32),
        ],
    )(_deg_body)
    seg_kernel = functools.partial(
        pl.kernel,
        out_type=jax.ShapeDtypeStruct((NC, N, D), jnp.float32),
        mesh=mesh,
        compiler_params=sc_params,
        scratch_types=[
            pltpu.VMEM_SHARED((N_ACC, D), jnp.float32),
            pltpu.VMEM((CT,), jnp.int32),
            pltpu.VMEM((CT,), jnp.int32),
        ]
        + [pltpu.VMEM((CH,), jnp.int32) for _ in range(2 * NSLOT)]
        + [pltpu.VMEM((CH, D), jnp.float32) for _ in range(NSLOT)]
        + [pltpu.SemaphoreType.DMA for _ in range(3 * NSLOT)],
    )(_seg_body)
    return deg_kernel, seg_kernel


def _deg_body(dst_hbm, out_hbm, hist, dstv):
    cid = lax.axis_index("c")
    sid = lax.axis_index("s")
    wid = sid * NC + cid

    zeros16 = jnp.zeros((16,), jnp.float32)

    def zbody(i, c):
        hist[pl.ds(i * 16, 16)] = zeros16
        return c

    lax.fori_loop(0, N_HIST // 16, zbody, 0)

    off = pl.multiple_of(wid * EPW_DEG, 8)
    pltpu.sync_copy(dst_hbm.at[pl.ds(off, EPW_DEG)], dstv)

    ones16 = jnp.ones((16,), jnp.float32)

    def body(i, c):
        idx = dstv[pl.ds(i * 16, 16)]
        plsc.addupdate_scatter(hist, [idx], ones16)
        return c

    lax.fori_loop(0, EPW_DEG // 16, body, 0)
    pltpu.sync_copy(hist, out_hbm.at[wid])


def _seg_body(xs_hbm, srcp_hbm, dstp_hbm, srct_hbm, dstt_hbm, out_hbm,
              acc, srct, dstt, *rest):
    sidx = rest[:NSLOT]
    didx = rest[NSLOT : 2 * NSLOT]
    rows = rest[2 * NSLOT : 3 * NSLOT]
    isem = rest[3 * NSLOT : 4 * NSLOT]
    gsem = rest[4 * NSLOT : 5 * NSLOT]
    ssem = rest[5 * NSLOT : 6 * NSLOT]
    cid = lax.axis_index("c")
    sid = lax.axis_index("s")
    wid = sid * NC + cid

    r0 = pl.multiple_of(sid * ROWS_PT, 8)
    pltpu.sync_copy(xs_hbm.at[pl.ds(r0, ROWS_PT)], acc.at[pl.ds(r0, ROWS_PT)])

    @pl.when(sid == 0)
    def _():
        t0 = NS * ROWS_PT
        pltpu.sync_copy(
            xs_hbm.at[pl.ds(t0, ROW_TAIL)], acc.at[pl.ds(t0, ROW_TAIL)]
        )

    pltpu.sync_copy(srct_hbm.at[wid], srct)
    pltpu.sync_copy(dstt_hbm.at[wid], dstt)
    plsc.subcore_barrier()

    def fire_idx(g, b):
        pltpu.async_copy(srcp_hbm.at[wid, g], sidx[b], isem[b])
        pltpu.async_copy(dstp_hbm.at[wid, g], didx[b], isem[b])

    def wait_idx(g, b):
        pltpu.make_async_copy(srcp_hbm.at[wid, g], sidx[b], isem[b]).wait()
        pltpu.make_async_copy(dstp_hbm.at[wid, g], didx[b], isem[b]).wait()

    def fire_gather(b):
        pltpu.async_copy(xs_hbm.at[sidx[b]], rows[b], gsem[b])

    def wait_gather(b):
        pltpu.make_async_copy(xs_hbm.at[sidx[b]], rows[b], gsem[b]).wait()

    def fire_scatter(b):
        pltpu.async_copy(rows[b], acc.at[didx[b]], ssem[b], add=True)

    def wait_scatter(b):
        pltpu.make_async_copy(rows[b], acc.at[didx[b]], ssem[b]).wait()

    def step(g, pos):
        b_a = pos % NSLOT
        if pos >= NSLOT:
            wait_scatter(b_a)
        fire_idx(g, b_a)
        if pos >= 1:
            b_b = (pos - 1) % NSLOT
            wait_idx(g - 1, b_b)
            fire_gather(b_b)
        if pos >= LAG:
            b_c = (pos - LAG) % NSLOT
            wait_gather(b_c)
            fire_scatter(b_c)

    for p in range(NSLOT):
        step(p, p)

    KS = (NCHUNK - NSLOT) // NSLOT

    def blk(j, c):
        g0 = NSLOT + NSLOT * j
        for b in range(NSLOT):
            step(g0 + b, NSLOT + b)
        return c

    lax.fori_loop(0, KS, blk, 0)

    for g in range(NSLOT + NSLOT * KS, NCHUNK):
        step(g, NSLOT + g % NSLOT)
    for g in range(NCHUNK, NCHUNK + LAG):
        b_b = (g - 1) % NSLOT
        if g - 1 < NCHUNK:
            wait_idx(g - 1, b_b)
            fire_gather(b_b)
        b_c = (g - LAG) % NSLOT
        if g - LAG < NCHUNK:
            wait_gather(b_c)
            fire_scatter(b_c)
    for c in range(NCHUNK - NSLOT, NCHUNK):
        wait_scatter(c % NSLOT)
    pltpu.async_copy(xs_hbm.at[srct], rows[0].at[pl.ds(0, CT)], gsem[0]).wait()
    pltpu.sync_copy(rows[0].at[pl.ds(0, CT)], acc.at[dstt], add=True)
    plsc.subcore_barrier()

    pltpu.sync_copy(
        acc.at[pl.ds(r0, ROWS_PT)], out_hbm.at[cid, pl.ds(r0, ROWS_PT)]
    )

    @pl.when(sid == 0)
    def _():
        t0 = NS * ROWS_PT
        pltpu.sync_copy(
            acc.at[pl.ds(t0, ROW_TAIL)], out_hbm.at[cid, pl.ds(t0, ROW_TAIL)]
        )


def _dinv_body(hists_ref, out_ref):
    deg = jnp.sum(hists_ref[...], axis=0) + 1.0
    out_ref[...] = lax.rsqrt(deg)


_dinv_call = pl.pallas_call(
    _dinv_body,
    out_shape=jax.ShapeDtypeStruct((N_HIST // 128, 128), jnp.float32),
)

BM = 2000


def _mm_body(x_ref, w_ref, dinv_ref, o_ref):
    xw = jnp.dot(x_ref[...], w_ref[...], preferred_element_type=jnp.float32)
    o_ref[...] = xw * dinv_ref[...]


_mm_call = pl.pallas_call(
    _mm_body,
    grid=(N // BM,),
    in_specs=[
        pl.BlockSpec((BM, D), lambda i: (i, 0)),
        pl.BlockSpec((D, D), lambda i: (0, 0)),
        pl.BlockSpec((BM, 1), lambda i: (i, 0)),
    ],
    out_specs=pl.BlockSpec((BM, D), lambda i: (i, 0)),
    out_shape=jax.ShapeDtypeStruct((N, D), jnp.float32),
)


def _post_body(p0_ref, p1_ref, xs_ref, dinv_ref, xin_ref, b_ref, g_ref, beta_ref, o_ref):
    agg = p0_ref[...] + p1_ref[...] - xs_ref[...]
    h = agg * dinv_ref[...] + b_ref[...]
    h = jnp.maximum(h, 0.0) + xin_ref[...]
    mu = jnp.mean(h, axis=-1, keepdims=True)
    d = h - mu
    var = jnp.mean(d * d, axis=-1, keepdims=True)
    o_ref[...] = d * lax.rsqrt(var + 1e-5) * g_ref[...] + beta_ref[...]


_post_call = pl.pallas_call(
    _post_body,
    grid=(N // BM,),
    in_specs=[
        pl.BlockSpec((BM, D), lambda i: (i, 0)),
        pl.BlockSpec((BM, D), lambda i: (i, 0)),
        pl.BlockSpec((BM, D), lambda i: (i, 0)),
        pl.BlockSpec((BM, 1), lambda i: (i, 0)),
        pl.BlockSpec((BM, D), lambda i: (i, 0)),
        pl.BlockSpec((1, D), lambda i: (0, 0)),
        pl.BlockSpec((1, D), lambda i: (0, 0)),
        pl.BlockSpec((1, D), lambda i: (0, 0)),
    ],
    out_specs=pl.BlockSpec((BM, D), lambda i: (i, 0)),
    out_shape=jax.ShapeDtypeStruct((N, D), jnp.float32),
)


@jax.jit
def _run(x, edge_index, Ws, bs, gammas, betas):
    src = edge_index[0]
    dst = edge_index[1]
    srcw = src.reshape(NW, EPW_DEG)
    dstw = dst.reshape(NW, EPW_DEG)
    srcp = srcw[:, : NCHUNK * CH].reshape(NW, NCHUNK, CH)
    dstp = dstw[:, : NCHUNK * CH].reshape(NW, NCHUNK, CH)
    srct = srcw[:, NCHUNK * CH :]
    dstt = dstw[:, NCHUNK * CH :]

    deg_kernel, seg_kernel = _sc_kernels()
    hists = deg_kernel(dst)
    dinv2d = _dinv_call(hists.reshape(NW, N_HIST // 128, 128))
    dinv_col = dinv2d.reshape(-1)[:N][:, None]

    for i in range(3):
        xs = _mm_call(x, Ws[i], dinv_col)
        parts = seg_kernel(xs, srcp, dstp, srct, dstt)
        x = _post_call(
            parts[0], parts[1], xs, dinv_col, x,
            bs[i][None, :], gammas[i][None, :], betas[i][None, :],
        )
    return x


def kernel(x, edge_index, Ws, bs, gammas, betas):
    return _run(x, edge_index, Ws, bs, gammas, betas)


# speedup vs baseline: 3.5936x; 1.0321x over previous
import functools

import jax
import jax.numpy as jnp
from jax import lax
from jax.experimental import pallas as pl
from jax.experimental.pallas import tpu as pltpu
from jax.experimental.pallas import tpu_sc as plsc

N = 10000
D = 128
E = 320000
NC = 2
NS = 16
NW = NC * NS

CH = 64
EPW_DEG = E // NW
NCHUNK = EPW_DEG // CH
CT = EPW_DEG - NCHUNK * CH
NSLOT = 6
LAG = 4
ROWS_PT = 624
ROW_TAIL = N - NS * ROWS_PT
N_ACC = N
N_HIST = 10240

@functools.cache
def _sc_kernels():
    mesh = plsc.VectorSubcoreMesh(
        core_axis_name="c", subcore_axis_name="s", num_cores=NC, num_subcores=NS
    )
    sc_params = pltpu.CompilerParams(needs_layout_passes=False)
    deg_kernel = functools.partial(
        pl.kernel,
        out_type=jax.ShapeDtypeStruct((NW, N_HIST), jnp.float32),
        mesh=mesh,
        compiler_params=sc_params,
        scratch_types=[
            pltpu.VMEM((N_HIST,), jnp.float32),
            pltpu.VMEM((EPW_DEG,), jnp.int32),
        ],
    )(_deg_body)
    seg_kernel = functools.partial(
        pl.kernel,
        out_type=jax.ShapeDtypeStruct((NC, N, D), jnp.float32),
        mesh=mesh,
        compiler_params=sc_params,
        scratch_types=[
            pltpu.VMEM_SHARED((N_ACC, D), jnp.float32),
            pltpu.VMEM((CT,), jnp.int32),
            pltpu.VMEM((CT,), jnp.int32),
        ]
        + [pltpu.VMEM((CH,), jnp.int32) for _ in range(2 * NSLOT)]
        + [pltpu.VMEM((CH, D), jnp.float32) for _ in range(NSLOT)]
        + [pltpu.SemaphoreType.DMA for _ in range(3 * NSLOT)],
    )(_seg_body)
    return deg_kernel, seg_kernel


def _deg_body(dst_hbm, out_hbm, hist, dstv):
    cid = lax.axis_index("c")
    sid = lax.axis_index("s")
    wid = sid * NC + cid

    zeros16 = jnp.zeros((16,), jnp.float32)

    def zbody(i, c):
        hist[pl.ds(i * 16, 16)] = zeros16
        return c

    lax.fori_loop(0, N_HIST // 16, zbody, 0)

    off = pl.multiple_of(wid * EPW_DEG, 8)
    pltpu.sync_copy(dst_hbm.at[pl.ds(off, EPW_DEG)], dstv)

    ones16 = jnp.ones((16,), jnp.float32)

    def body(i, c):
        idx = dstv[pl.ds(i * 16, 16)]
        plsc.addupdate_scatter(hist, [idx], ones16)
        return c

    lax.fori_loop(0, EPW_DEG // 16, body, 0)
    pltpu.sync_copy(hist, out_hbm.at[wid])


def _seg_body(xs_hbm, srcp_hbm, dstp_hbm, srct_hbm, dstt_hbm, out_hbm,
              acc, srct, dstt, *rest):
    sidx = rest[:NSLOT]
    didx = rest[NSLOT : 2 * NSLOT]
    rows = rest[2 * NSLOT : 3 * NSLOT]
    isem = rest[3 * NSLOT : 4 * NSLOT]
    gsem = rest[4 * NSLOT : 5 * NSLOT]
    ssem = rest[5 * NSLOT : 6 * NSLOT]
    cid = lax.axis_index("c")
    sid = lax.axis_index("s")
    wid = sid * NC + cid

    r0 = pl.multiple_of(sid * ROWS_PT, 8)
    pltpu.sync_copy(xs_hbm.at[pl.ds(r0, ROWS_PT)], acc.at[pl.ds(r0, ROWS_PT)])

    @pl.when(sid == 0)
    def _():
        t0 = NS * ROWS_PT
        pltpu.sync_copy(
            xs_hbm.at[pl.ds(t0, ROW_TAIL)], acc.at[pl.ds(t0, ROW_TAIL)]
        )

    pltpu.sync_copy(srct_hbm.at[wid], srct)
    pltpu.sync_copy(dstt_hbm.at[wid], dstt)
    plsc.subcore_barrier()

    def fire_idx(g, b):
        pltpu.async_copy(srcp_hbm.at[wid, g], sidx[b], isem[b])
        pltpu.async_copy(dstp_hbm.at[wid, g], didx[b], isem[b])

    def wait_idx(g, b):
        pltpu.make_async_copy(srcp_hbm.at[wid, g], sidx[b], isem[b]).wait()
        pltpu.make_async_copy(dstp_hbm.at[wid, g], didx[b], isem[b]).wait()

    def fire_gather(b):
        pltpu.async_copy(xs_hbm.at[sidx[b]], rows[b], gsem[b])

    def wait_gather(b):
        pltpu.make_async_copy(xs_hbm.at[sidx[b]], rows[b], gsem[b]).wait()

    def fire_scatter(b):
        pltpu.async_copy(rows[b], acc.at[didx[b]], ssem[b], add=True)

    def wait_scatter(b):
        pltpu.make_async_copy(rows[b], acc.at[didx[b]], ssem[b]).wait()

    def step(g, pos):
        b_a = pos % NSLOT
        if pos >= NSLOT:
            wait_scatter(b_a)
        fire_idx(g, b_a)
        if pos >= 1:
            b_b = (pos - 1) % NSLOT
            wait_idx(g - 1, b_b)
            fire_gather(b_b)
        if pos >= LAG:
            b_c = (pos - LAG) % NSLOT
            wait_gather(b_c)
            fire_scatter(b_c)

    for p in range(NSLOT):
        step(p, p)

    KS = (NCHUNK - NSLOT) // NSLOT

    def blk(j, c):
        g0 = NSLOT + NSLOT * j
        for b in range(NSLOT):
            step(g0 + b, NSLOT + b)
        return c

    lax.fori_loop(0, KS, blk, 0)

    for g in range(NSLOT + NSLOT * KS, NCHUNK):
        step(g, NSLOT + g % NSLOT)
    for g in range(NCHUNK, NCHUNK + LAG):
        b_b = (g - 1) % NSLOT
        if g - 1 < NCHUNK:
            wait_idx(g - 1, b_b)
            fire_gather(b_b)
        b_c = (g - LAG) % NSLOT
        if g - LAG < NCHUNK:
            wait_gather(b_c)
            fire_scatter(b_c)
    for c in range(NCHUNK - NSLOT, NCHUNK):
        wait_scatter(c % NSLOT)
    pltpu.async_copy(xs_hbm.at[srct], rows[0].at[pl.ds(0, CT)], gsem[0]).wait()
    pltpu.sync_copy(rows[0].at[pl.ds(0, CT)], acc.at[dstt], add=True)
    plsc.subcore_barrier()

    pltpu.sync_copy(
        acc.at[pl.ds(r0, ROWS_PT)], out_hbm.at[cid, pl.ds(r0, ROWS_PT)]
    )

    @pl.when(sid == 0)
    def _():
        t0 = NS * ROWS_PT
        pltpu.sync_copy(
            acc.at[pl.ds(t0, ROW_TAIL)], out_hbm.at[cid, pl.ds(t0, ROW_TAIL)]
        )


def _dinv_body(hists_ref, out_ref):
    deg = jnp.sum(hists_ref[...], axis=0) + 1.0
    out_ref[...] = lax.rsqrt(deg)


_dinv_call = pl.pallas_call(
    _dinv_body,
    out_shape=jax.ShapeDtypeStruct((N_HIST // 128, 128), jnp.float32),
)

BM = 2000


def _mm_body(x_ref, w_ref, dinv_ref, o_ref):
    xw = jnp.dot(x_ref[...], w_ref[...], preferred_element_type=jnp.float32)
    o_ref[...] = xw * dinv_ref[...]


_mm_call = pl.pallas_call(
    _mm_body,
    grid=(N // BM,),
    in_specs=[
        pl.BlockSpec((BM, D), lambda i: (i, 0)),
        pl.BlockSpec((D, D), lambda i: (0, 0)),
        pl.BlockSpec((BM, 1), lambda i: (i, 0)),
    ],
    out_specs=pl.BlockSpec((BM, D), lambda i: (i, 0)),
    out_shape=jax.ShapeDtypeStruct((N, D), jnp.float32),
)


def _post_body(p0_ref, p1_ref, xs_ref, dinv_ref, xin_ref, b_ref, g_ref, beta_ref, o_ref):
    agg = p0_ref[...] + p1_ref[...] - xs_ref[...]
    h = agg * dinv_ref[...] + b_ref[...]
    h = jnp.maximum(h, 0.0) + xin_ref[...]
    mu = jnp.mean(h, axis=-1, keepdims=True)
    d = h - mu
    var = jnp.mean(d * d, axis=-1, keepdims=True)
    o_ref[...] = d * lax.rsqrt(var + 1e-5) * g_ref[...] + beta_ref[...]


def _fused_body(p0_ref, p1_ref, xs_ref, dinv_ref, xin_ref, b_ref, g_ref,
                beta_ref, w_ref, ox_ref, oxs_ref):
    agg = p0_ref[...] + p1_ref[...] - xs_ref[...]
    h = agg * dinv_ref[...] + b_ref[...]
    h = jnp.maximum(h, 0.0) + xin_ref[...]
    mu = jnp.mean(h, axis=-1, keepdims=True)
    d = h - mu
    var = jnp.mean(d * d, axis=-1, keepdims=True)
    xnew = d * lax.rsqrt(var + 1e-5) * g_ref[...] + beta_ref[...]
    ox_ref[...] = xnew
    oxs_ref[...] = (
        jnp.dot(xnew, w_ref[...], preferred_element_type=jnp.float32)
        * dinv_ref[...]
    )


_fused_call = pl.pallas_call(
    _fused_body,
    grid=(N // BM,),
    in_specs=[
        pl.BlockSpec((BM, D), lambda i: (i, 0)),
        pl.BlockSpec((BM, D), lambda i: (i, 0)),
        pl.BlockSpec((BM, D), lambda i: (i, 0)),
        pl.BlockSpec((BM, 1), lambda i: (i, 0)),
        pl.BlockSpec((BM, D), lambda i: (i, 0)),
        pl.BlockSpec((1, D), lambda i: (0, 0)),
        pl.BlockSpec((1, D), lambda i: (0, 0)),
        pl.BlockSpec((1, D), lambda i: (0, 0)),
        pl.BlockSpec((D, D), lambda i: (0, 0)),
    ],
    out_specs=[
        pl.BlockSpec((BM, D), lambda i: (i, 0)),
        pl.BlockSpec((BM, D), lambda i: (i, 0)),
    ],
    out_shape=[
        jax.ShapeDtypeStruct((N, D), jnp.float32),
        jax.ShapeDtypeStruct((N, D), jnp.float32),
    ],
)

_post_call = pl.pallas_call(
    _post_body,
    grid=(N // BM,),
    in_specs=[
        pl.BlockSpec((BM, D), lambda i: (i, 0)),
        pl.BlockSpec((BM, D), lambda i: (i, 0)),
        pl.BlockSpec((BM, D), lambda i: (i, 0)),
        pl.BlockSpec((BM, 1), lambda i: (i, 0)),
        pl.BlockSpec((BM, D), lambda i: (i, 0)),
        pl.BlockSpec((1, D), lambda i: (0, 0)),
        pl.BlockSpec((1, D), lambda i: (0, 0)),
        pl.BlockSpec((1, D), lambda i: (0, 0)),
    ],
    out_specs=pl.BlockSpec((BM, D), lambda i: (i, 0)),
    out_shape=jax.ShapeDtypeStruct((N, D), jnp.float32),
)


@jax.jit
def _run(x, edge_index, Ws, bs, gammas, betas):
    src = edge_index[0]
    dst = edge_index[1]
    srcw = src.reshape(NW, EPW_DEG)
    dstw = dst.reshape(NW, EPW_DEG)
    srcp = srcw[:, : NCHUNK * CH].reshape(NW, NCHUNK, CH)
    dstp = dstw[:, : NCHUNK * CH].reshape(NW, NCHUNK, CH)
    srct = srcw[:, NCHUNK * CH :]
    dstt = dstw[:, NCHUNK * CH :]

    deg_kernel, seg_kernel = _sc_kernels()
    hists = deg_kernel(dst)
    dinv2d = _dinv_call(hists.reshape(NW, N_HIST // 128, 128))
    dinv_col = dinv2d.reshape(-1)[:N][:, None]

    xs = _mm_call(x, Ws[0], dinv_col)
    for i in range(2):
        parts = seg_kernel(xs, srcp, dstp, srct, dstt)
        x, xs = _fused_call(
            parts[0], parts[1], xs, dinv_col, x,
            bs[i][None, :], gammas[i][None, :], betas[i][None, :], Ws[i + 1],
        )
    parts = seg_kernel(xs, srcp, dstp, srct, dstt)
    x = _post_call(
        parts[0], parts[1], xs, dinv_col, x,
        bs[2][None, :], gammas[2][None, :], betas[2][None, :],
    )
    return x


def kernel(x, edge_index, Ws, bs, gammas, betas):
    return _run(x, edge_index, Ws, bs, gammas, betas)
